# baseline (speedup 1.0000x reference)
import functools

import jax
import jax.numpy as jnp
from jax import lax
from jax.experimental import pallas as pl
from jax.experimental.pallas import tpu as pltpu
from jax.experimental.pallas import tpu_sc as plsc

NC = 2
NS = 16
NW = NC * NS
CH = 80
NQ = 2
Q = 64

_SC_PARAMS = pltpu.CompilerParams(use_tc_tiling_on_sc=False)


def _sc_mesh():
    return plsc.VectorSubcoreMesh(core_axis_name="c", subcore_axis_name="s")


def _degree_kernel(np_, e):
    epw = e // NW
    nch = epw // CH
    rps = np_ // NS
    assert nch % 5 == 0

    @functools.partial(
        pl.kernel,
        mesh=_sc_mesh(),
        compiler_params=_SC_PARAMS,
        out_type=jax.ShapeDtypeStruct((NC * np_, 16), jnp.float32),
        scratch_types=[
            pltpu.VMEM((nch, CH), jnp.int32),
            pltpu.VMEM((CH, 16), jnp.float32),
            pltpu.VMEM_SHARED((np_, 16), jnp.float32),
            pltpu.SemaphoreType.DMA,
        ],
    )
    def deg(row2_hbm, ones_hbm, zrow_hbm, out_hbm, ridx2, ones_v, acc, sem):
        c = lax.axis_index("c")
        s = lax.axis_index("s")
        w = c * NS + s
        pltpu.sync_copy(row2_hbm.at[pl.ds(w * nch, nch)], ridx2)
        pltpu.sync_copy(zrow_hbm, acc.at[pl.ds(s * rps, rps)])
        pltpu.sync_copy(ones_hbm, ones_v)
        plsc.subcore_barrier()

        def body(k, carry):
            for j in range(5):
                pltpu.async_copy(ones_v, acc.at[ridx2.at[5 * k + j]], sem,
                                 add=True)
            for j in range(5):
                pltpu.make_async_copy(ones_v, acc.at[ridx2.at[5 * k + j]],
                                      sem).wait()
            return carry

        lax.fori_loop(0, nch // 5, body, 0)
        plsc.subcore_barrier()
        pltpu.sync_copy(acc.at[pl.ds(s * rps, rps)],
                        out_hbm.at[pl.ds(c * np_ + s * rps, rps)])

    return deg


def _spmv_kernel(np_, e):
    epw = e // NW
    nch = epw // CH
    rps = np_ // NS
    nk = (nch - 1) // 4
    assert nch == 4 * nk + 1

    @functools.partial(
        pl.kernel,
        mesh=_sc_mesh(),
        compiler_params=_SC_PARAMS,
        out_type=jax.ShapeDtypeStruct((NC * NQ * np_, Q), jnp.float32),
        scratch_types=[
            pltpu.VMEM((nch, CH), jnp.int32),
            pltpu.VMEM((nch, CH), jnp.int32),
            [pltpu.VMEM((CH, Q), jnp.float32) for _ in range(4)],
            pltpu.VMEM_SHARED((np_, Q), jnp.float32),
            [pltpu.SemaphoreType.DMA for _ in range(4)],
            [pltpu.SemaphoreType.DMA for _ in range(4)],
        ],
    )
    def spmv(u0_hbm, u1_hbm, row2_hbm, col2_hbm, zrow_hbm,
             out_hbm, ridx2, cidx2, bufs, acc, gsems, ssems):
        c = lax.axis_index("c")
        s = lax.axis_index("s")
        w = c * NS + s
        pltpu.sync_copy(row2_hbm.at[pl.ds(w * nch, nch)], ridx2)
        pltpu.sync_copy(col2_hbm.at[pl.ds(w * nch, nch)], cidx2)

        for q, u_hbm in enumerate((u0_hbm, u1_hbm)):
            pltpu.sync_copy(zrow_hbm, acc.at[pl.ds(s * rps, rps)])
            plsc.subcore_barrier()

            def start_g(g, b, u=u_hbm):
                pltpu.async_copy(u.at[ridx2.at[g]], bufs[b], gsems[b])

            def wait_g(g, b, u=u_hbm):
                pltpu.make_async_copy(u.at[ridx2.at[g]], bufs[b],
                                      gsems[b]).wait()

            def start_s(g, b):
                pltpu.async_copy(bufs[b], acc.at[cidx2.at[g]], ssems[b],
                                 add=True)

            def wait_s(g, b):
                pltpu.make_async_copy(bufs[b], acc.at[cidx2.at[g]],
                                      ssems[b]).wait()

            for b in range(3):
                start_g(b, b)
            for j in range(4):
                if j > 0:
                    wait_s(j - 1, (j + 3) % 4)
                start_g(j + 3, (j + 3) % 4)
                wait_g(j, j)
                start_s(j, j)

            def body(k, carry):
                for j in range(4):
                    g = 4 * k + j
                    bi = (j + 3) % 4
                    wait_s(g - 1, bi)
                    start_g(g + 3, bi)
                    wait_g(g, j)
                    start_s(g, j)
                return carry

            lax.fori_loop(1, nk - 1, body, 0)

            g0 = 4 * (nk - 1)
            for j in range(4):
                bi = (j + 3) % 4
                wait_s(g0 + j - 1, bi)
                if g0 + j + 3 <= nch - 1:
                    start_g(g0 + j + 3, bi)
                wait_g(g0 + j, j)
                start_s(g0 + j, j)

            wait_g(nch - 1, 0)
            start_s(nch - 1, 0)
            wait_s(nch - 2, 3)
            wait_s(nch - 1, 0)
            plsc.subcore_barrier()
            pltpu.sync_copy(
                acc.at[pl.ds(s * rps, rps)],
                out_hbm.at[pl.ds((c * NQ + q) * np_ + s * rps, rps)])
            plsc.subcore_barrier()

    return spmv


def _full_spec(shape):
    nd = len(shape)
    return pl.BlockSpec(shape, lambda i, _nd=nd: (0,) * _nd)


def _tc_call(body, out_widths, bn, np_, d, *args):
    grid = np_ // bn
    in_specs = []
    for a in args:
        if a.ndim == 4:
            in_specs.append(
                pl.BlockSpec((NC, NQ, bn, Q), lambda i: (0, 0, i, 0)))
        elif a.shape[0] == np_:
            in_specs.append(
                pl.BlockSpec((bn, a.shape[1]), lambda i: (i, 0)))
        else:
            in_specs.append(_full_spec(a.shape))
    out_shapes = tuple(
        jax.ShapeDtypeStruct((np_, w), jnp.float32) for w in out_widths)
    out_specs = tuple(
        pl.BlockSpec((bn, w), lambda i: (i, 0)) for w in out_widths)
    return pl.pallas_call(
        body,
        grid=(grid,),
        in_specs=in_specs,
        out_shape=out_shapes,
        out_specs=out_specs,
    )(*args)


def _tc_dis(dparts, bn, np_):
    def body(dref, dis16_ref):
        deg = dref[0, :, 0:1] + dref[1, :, 0:1]
        dis = jnp.where(deg > 0.0, lax.rsqrt(jnp.maximum(deg, 1e-30)), 0.0)
        dis16_ref[...] = jnp.broadcast_to(dis, (bn, 16))

    return pl.pallas_call(
        body,
        grid=(np_ // bn,),
        in_specs=[pl.BlockSpec((NC, bn, 16), lambda i: (0, i, 0))],
        out_shape=jax.ShapeDtypeStruct((np_, 16), jnp.float32),
        out_specs=pl.BlockSpec((bn, 16), lambda i: (i, 0)),
    )(dparts)


def _tc_pre(dis16, x, w0, bn, np_, d):
    def body(dref, xref, wref, ul_ref, uh_ref, acc_ref):
        xb = xref[...]
        u = dref[:, 0:1] * xb
        ul_ref[...] = u[:, :Q]
        uh_ref[...] = u[:, Q:]
        acc_ref[...] = jnp.dot(xb, wref[...],
                               preferred_element_type=jnp.float32)
    return _tc_call(body, (Q, Q, d), bn, np_, d, dis16, x, w0)


def _tc_step(sparts, dis16, tp, acc, wk, wb, bk, ca, cb, fb, bn, np_, d):
    def body(sref, dref, tref, aref, wkref, wbref, bref, caref, cbref,
             fbref, z_ref, ul_ref, uh_ref, tc_ref, acc_ref):
        dis = dref[:, 0:1]
        s = jnp.concatenate(
            [sref[0, q] + sref[1, q] for q in range(NQ)], axis=1)
        t_new = caref[...] * dis * s + cbref[...] * tref[...]
        z = aref[...] + jnp.dot(t_new, wkref[...],
                                preferred_element_type=jnp.float32) + bref[...]
        h = jnp.maximum(z, 0.0)
        fb = fbref[...] > 0.0
        fed = jnp.where(fb, h, t_new)
        z_ref[...] = z
        u = dis * fed
        ul_ref[...] = u[:, :Q]
        uh_ref[...] = u[:, Q:]
        tc_ref[...] = fed
        acc_ref[...] = jnp.where(
            fb, jnp.dot(h, wbref[...], preferred_element_type=jnp.float32), z)
    return _tc_call(body, (d, Q, Q, d, d), bn, np_, d,
                    sparts, dis16, tp, acc, wk, wb, bk, ca, cb, fb)


def kernel(x, edge_index, W1, b1, W2, b2):
    n, d = x.shape
    e = edge_index.shape[1]
    np_ = 10240
    bn = 1024
    rps = np_ // NS
    row2 = jnp.reshape(edge_index[0], (e // CH, CH))
    col2 = jnp.reshape(edge_index[1], (e // CH, CH))

    xp = jnp.pad(x, ((0, np_ - n), (0, 0)))
    zrow_q = jnp.zeros((rps, Q), jnp.float32)
    zrow_16 = jnp.zeros((rps, 16), jnp.float32)
    ones16 = jnp.ones((CH, 16), jnp.float32)

    def bcast(v):
        return jnp.broadcast_to(jnp.float32(v), (1, d))

    zb = jnp.zeros((1, d), jnp.float32)
    wks = jnp.stack((W1[1], W1[2], W2[1], W2[2]))
    wbs = jnp.stack((W2[0], W2[0], W2[0], W2[0]))
    bks = jnp.stack((zb, jnp.reshape(b1, (1, d)), zb, jnp.reshape(b2, (1, d))))
    cas = jnp.stack((bcast(-1.0), bcast(-2.0), bcast(-1.0), bcast(-2.0)))
    cbs = jnp.stack((bcast(0.0), bcast(-1.0), bcast(0.0), bcast(-1.0)))
    fbs = jnp.stack((bcast(0.0), bcast(1.0), bcast(0.0), bcast(0.0)))

    deg_fn = _degree_kernel(np_, e)
    spmv_fn = _spmv_kernel(np_, e)

    def spmv(ul, uh):
        parts = spmv_fn(ul, uh, row2, col2, zrow_q)
        return jnp.reshape(parts, (NC, NQ, np_, Q))

    dparts = jnp.reshape(deg_fn(row2, ones16, zrow_16), (NC, np_, 16))
    dis16 = _tc_dis(dparts, bn, np_)
    u0l, u0h, acc0 = _tc_pre(dis16, xp, W1[0], bn, np_, d)

    def half_step(state, wbr):
        ul, uh, tp, tc_, acc, _ = state
        wk, wb, bk, ca, cb, fb = wbr
        s = spmv(ul, uh)
        z, ul2, uh2, tc2, acc2 = _tc_step(s, dis16, tp, acc, wk, wb, bk,
                                          ca, cb, fb, bn, np_, d)
        return (ul2, uh2, tc_, tc2, acc2, z), None

    state0 = (u0l, u0h, jnp.zeros_like(xp), xp, acc0, jnp.zeros_like(xp))
    state, _ = lax.scan(half_step, state0, (wks, wbs, bks, cas, cbs, fbs))
    return state[5][:n]


# speedup vs baseline: 1.0706x; 1.0706x over previous
import functools

import jax
import jax.numpy as jnp
from jax import lax
from jax.experimental import pallas as pl
from jax.experimental.pallas import tpu as pltpu
from jax.experimental.pallas import tpu_sc as plsc

NC = 2
NS = 16
NW = NC * NS
CH = 80
NQ = 2
Q = 64

_SC_PARAMS = pltpu.CompilerParams(use_tc_tiling_on_sc=False)


def _sc_mesh():
    return plsc.VectorSubcoreMesh(core_axis_name="c", subcore_axis_name="s")


def _degree_kernel(np_, e):
    epw = e // NW
    nch = epw // CH
    rps = np_ // NS
    assert nch % 5 == 0

    @functools.partial(
        pl.kernel,
        mesh=_sc_mesh(),
        compiler_params=_SC_PARAMS,
        out_type=jax.ShapeDtypeStruct((NC * np_, 16), jnp.float32),
        scratch_types=[
            pltpu.VMEM((nch, CH), jnp.int32),
            pltpu.VMEM((CH, 16), jnp.float32),
            pltpu.VMEM_SHARED((np_, 16), jnp.float32),
            pltpu.SemaphoreType.DMA,
        ],
    )
    def deg(row2_hbm, ones_hbm, zrow_hbm, out_hbm, ridx2, ones_v, acc, sem):
        c = lax.axis_index("c")
        s = lax.axis_index("s")
        w = c * NS + s
        pltpu.sync_copy(row2_hbm.at[pl.ds(w * nch, nch)], ridx2)
        pltpu.sync_copy(zrow_hbm, acc.at[pl.ds(s * rps, rps)])
        pltpu.sync_copy(ones_hbm, ones_v)
        plsc.subcore_barrier()

        def body(k, carry):
            for j in range(5):
                pltpu.async_copy(ones_v, acc.at[ridx2.at[5 * k + j]], sem,
                                 add=True)
            for j in range(5):
                pltpu.make_async_copy(ones_v, acc.at[ridx2.at[5 * k + j]],
                                      sem).wait()
            return carry

        lax.fori_loop(0, nch // 5, body, 0)
        plsc.subcore_barrier()
        pltpu.sync_copy(acc.at[pl.ds(s * rps, rps)],
                        out_hbm.at[pl.ds(c * np_ + s * rps, rps)])

    return deg


def _spmv_kernel(np_, e):
    epw = e // NW
    nch = epw // CH
    rps = np_ // NS
    nk = (nch - 1) // 4
    assert nch == 4 * nk + 1

    @functools.partial(
        pl.kernel,
        mesh=_sc_mesh(),
        compiler_params=_SC_PARAMS,
        out_type=jax.ShapeDtypeStruct((NC * NQ * np_, Q), jnp.float32),
        scratch_types=[
            pltpu.VMEM((nch, CH), jnp.int32),
            pltpu.VMEM((nch, CH), jnp.int32),
            [pltpu.VMEM((CH, Q), jnp.float32) for _ in range(4)],
            pltpu.VMEM_SHARED((np_, Q), jnp.float32),
            [pltpu.SemaphoreType.DMA for _ in range(4)],
            [pltpu.SemaphoreType.DMA for _ in range(4)],
        ],
    )
    def spmv(u0_hbm, u1_hbm, row2_hbm, col2_hbm, zrow_hbm,
             out_hbm, ridx2, cidx2, bufs, acc, gsems, ssems):
        c = lax.axis_index("c")
        s = lax.axis_index("s")
        w = c * NS + s
        pltpu.sync_copy(row2_hbm.at[pl.ds(w * nch, nch)], ridx2)
        pltpu.sync_copy(col2_hbm.at[pl.ds(w * nch, nch)], cidx2)

        for q, u_hbm in enumerate((u0_hbm, u1_hbm)):
            pltpu.sync_copy(zrow_hbm, acc.at[pl.ds(s * rps, rps)])
            plsc.subcore_barrier()

            def start_g(g, b, u=u_hbm):
                pltpu.async_copy(u.at[ridx2.at[g]], bufs[b], gsems[b])

            def wait_g(g, b, u=u_hbm):
                pltpu.make_async_copy(u.at[ridx2.at[g]], bufs[b],
                                      gsems[b]).wait()

            def start_s(g, b):
                pltpu.async_copy(bufs[b], acc.at[cidx2.at[g]], ssems[b],
                                 add=True)

            def wait_s(g, b):
                pltpu.make_async_copy(bufs[b], acc.at[cidx2.at[g]],
                                      ssems[b]).wait()

            for b in range(3):
                start_g(b, b)
            for j in range(4):
                if j > 0:
                    wait_s(j - 1, (j + 3) % 4)
                start_g(j + 3, (j + 3) % 4)
                wait_g(j, j)
                start_s(j, j)

            def body(k, carry):
                for j in range(4):
                    g = 4 * k + j
                    bi = (j + 3) % 4
                    wait_s(g - 1, bi)
                    start_g(g + 3, bi)
                    wait_g(g, j)
                    start_s(g, j)
                return carry

            lax.fori_loop(1, nk - 1, body, 0)

            g0 = 4 * (nk - 1)
            for j in range(4):
                bi = (j + 3) % 4
                wait_s(g0 + j - 1, bi)
                if g0 + j + 3 <= nch - 1:
                    start_g(g0 + j + 3, bi)
                wait_g(g0 + j, j)
                start_s(g0 + j, j)

            wait_g(nch - 1, 0)
            start_s(nch - 1, 0)
            wait_s(nch - 2, 3)
            wait_s(nch - 1, 0)
            plsc.subcore_barrier()
            pltpu.sync_copy(
                acc.at[pl.ds(s * rps, rps)],
                out_hbm.at[pl.ds((c * NQ + q) * np_ + s * rps, rps)])
            plsc.subcore_barrier()

    return spmv


def _full_spec(shape):
    nd = len(shape)
    return pl.BlockSpec(shape, lambda i, _nd=nd: (0,) * _nd)


def _tc_call(body, out_widths, bn, np_, d, *args):
    grid = np_ // bn
    in_specs = []
    for a in args:
        if a.ndim == 4:
            in_specs.append(
                pl.BlockSpec((NC, NQ, bn, Q), lambda i: (0, 0, i, 0)))
        elif a.ndim == 3 and a.shape[0] == NC:
            in_specs.append(
                pl.BlockSpec((NC, bn, a.shape[2]), lambda i: (0, i, 0)))
        elif a.shape[0] == np_:
            in_specs.append(
                pl.BlockSpec((bn, a.shape[1]), lambda i: (i, 0)))
        else:
            in_specs.append(_full_spec(a.shape))
    out_shapes = tuple(
        jax.ShapeDtypeStruct((np_, w), jnp.float32) for w in out_widths)
    out_specs = tuple(
        pl.BlockSpec((bn, w), lambda i: (i, 0)) for w in out_widths)
    return pl.pallas_call(
        body,
        grid=(grid,),
        in_specs=in_specs,
        out_shape=out_shapes,
        out_specs=out_specs,
    )(*args)


def _sum_parts(sref):
    return jnp.concatenate(
        [sref[0, q] + sref[1, q] for q in range(NQ)], axis=1)


def _store_halves(u, urefs):
    for q, uref in enumerate(urefs):
        uref[...] = u[:, q * Q:(q + 1) * Q]


def _tc_a(dparts, x, w0, bn, np_, d):
    def body(dref, xref, wref, dis16_ref, uq0, uq1, acc_ref):
        deg = dref[0, :, 0:1] + dref[1, :, 0:1]
        dis = jnp.where(deg > 0.0, lax.rsqrt(jnp.maximum(deg, 1e-30)), 0.0)
        dis16_ref[...] = jnp.broadcast_to(dis, (bn, 16))
        xb = xref[...]
        _store_halves(dis * xb, (uq0, uq1))
        acc_ref[...] = jnp.dot(xb, wref[...],
                               preferred_element_type=jnp.float32)
    return _tc_call(body, (16, Q, Q, d), bn, np_, d, dparts, x, w0)


def _tc_b(sparts, dis16, wk, acc, bn, np_, d):
    def body(sref, dref, wref, aref, uq0, uq1, acc_ref):
        dis = dref[:, 0:1]
        tx1 = -dis * _sum_parts(sref)
        _store_halves(dis * tx1, (uq0, uq1))
        acc_ref[...] = aref[...] + jnp.dot(
            tx1, wref[...], preferred_element_type=jnp.float32)
    return _tc_call(body, (Q, Q, d), bn, np_, d, sparts, dis16, wk, acc)


def _tc_e(sparts, dis16, tx0, acc, wk, b, rfl, bn, np_, d):
    def body(sref, dref, tref, aref, wref, bref, rref, out_ref):
        dis = dref[:, 0:1]
        tx2 = -2.0 * dis * _sum_parts(sref) - tref[...]
        o = aref[...] + jnp.dot(
            tx2, wref[...], preferred_element_type=jnp.float32) + bref[...]
        out_ref[...] = jnp.where(rref[...] > 0.0, jnp.maximum(o, 0.0), o)
    return _tc_call(body, (d,), bn, np_, d,
                    sparts, dis16, tx0, acc, wk, b, rfl)[0]


def kernel(x, edge_index, W1, b1, W2, b2):
    n, d = x.shape
    e = edge_index.shape[1]
    np_ = 10240
    bn = 1024
    rps = np_ // NS
    row2 = jnp.reshape(edge_index[0], (e // CH, CH))
    col2 = jnp.reshape(edge_index[1], (e // CH, CH))

    xp = jnp.pad(x, ((0, np_ - n), (0, 0)))
    zrow_q = jnp.zeros((rps, Q), jnp.float32)
    zrow_16 = jnp.zeros((rps, 16), jnp.float32)
    ones16 = jnp.ones((CH, 16), jnp.float32)
    ws = jnp.stack((W1, W2))
    bs = jnp.stack((jnp.reshape(b1, (1, d)), jnp.reshape(b2, (1, d))))
    rfls = jnp.stack((jnp.ones((1, d), jnp.float32),
                      jnp.zeros((1, d), jnp.float32)))

    deg_fn = _degree_kernel(np_, e)
    spmv_fn = _spmv_kernel(np_, e)

    def spmv(u0, u1):
        parts = spmv_fn(u0, u1, row2, col2, zrow_q)
        return jnp.reshape(parts, (NC, NQ, np_, Q))

    dparts = jnp.reshape(deg_fn(row2, ones16, zrow_16), (NC, np_, 16))

    def layer(x_in, wbr):
        wk, bk, rfl = wbr
        dis16, u0, u1, acc = _tc_a(dparts, x_in, wk[0], bn, np_, d)
        s1 = spmv(u0, u1)
        v0, v1, acc = _tc_b(s1, dis16, wk[1], acc, bn, np_, d)
        s2 = spmv(v0, v1)
        out = _tc_e(s2, dis16, x_in, acc, wk[2], bk, rfl, bn, np_, d)
        return out, None

    out, _ = lax.scan(layer, xp, (ws, bs, rfls))
    return out[:n]


# speedup vs baseline: 1.0750x; 1.0041x over previous
import functools

import jax
import jax.numpy as jnp
from jax import lax
from jax.experimental import pallas as pl
from jax.experimental.pallas import tpu as pltpu
from jax.experimental.pallas import tpu_sc as plsc

NC = 2
NS = 16
NW = NC * NS
CH = 80
NQ = 2
Q = 64

_SC_PARAMS = pltpu.CompilerParams(use_tc_tiling_on_sc=False)


def _sc_mesh():
    return plsc.VectorSubcoreMesh(core_axis_name="c", subcore_axis_name="s")


def _degree_kernel(np_, e):
    epw = e // NW
    nch = epw // CH
    rps = np_ // NS
    assert nch % 5 == 0

    @functools.partial(
        pl.kernel,
        mesh=_sc_mesh(),
        compiler_params=_SC_PARAMS,
        out_type=jax.ShapeDtypeStruct((NC * np_, 16), jnp.float32),
        scratch_types=[
            pltpu.VMEM((nch, CH), jnp.int32),
            pltpu.VMEM((CH, 16), jnp.float32),
            pltpu.VMEM_SHARED((np_, 16), jnp.float32),
            pltpu.SemaphoreType.DMA,
        ],
    )
    def deg(row2_hbm, ones_hbm, zrow_hbm, out_hbm, ridx2, ones_v, acc, sem):
        c = lax.axis_index("c")
        s = lax.axis_index("s")
        w = c * NS + s
        pltpu.sync_copy(row2_hbm.at[pl.ds(w * nch, nch)], ridx2)
        pltpu.sync_copy(zrow_hbm, acc.at[pl.ds(s * rps, rps)])
        pltpu.sync_copy(ones_hbm, ones_v)
        plsc.subcore_barrier()

        def body(k, carry):
            for j in range(5):
                pltpu.async_copy(ones_v, acc.at[ridx2.at[5 * k + j]], sem,
                                 add=True)
            for j in range(5):
                pltpu.make_async_copy(ones_v, acc.at[ridx2.at[5 * k + j]],
                                      sem).wait()
            return carry

        lax.fori_loop(0, nch // 5, body, 0)
        plsc.subcore_barrier()
        pltpu.sync_copy(acc.at[pl.ds(s * rps, rps)],
                        out_hbm.at[pl.ds(c * np_ + s * rps, rps)])

    return deg


def _spmv_kernel(np_, e):
    epw = e // NW
    nch = epw // CH
    rps = np_ // NS
    nk = (nch - 1) // 4
    assert nch == 4 * nk + 1

    @functools.partial(
        pl.kernel,
        mesh=_sc_mesh(),
        compiler_params=_SC_PARAMS,
        out_type=jax.ShapeDtypeStruct((NC * NQ * np_, Q), jnp.float32),
        scratch_types=[
            pltpu.VMEM((nch, CH), jnp.int32),
            pltpu.VMEM((nch, CH), jnp.int32),
            [pltpu.VMEM((CH, Q), jnp.float32) for _ in range(4)],
            pltpu.VMEM_SHARED((np_, Q), jnp.float32),
            [pltpu.SemaphoreType.DMA for _ in range(4)],
            [pltpu.SemaphoreType.DMA for _ in range(4)],
        ],
    )
    def spmv(u0_hbm, u1_hbm, row2_hbm, col2_hbm, zrow_hbm,
             out_hbm, ridx2, cidx2, bufs, acc, gsems, ssems):
        c = lax.axis_index("c")
        s = lax.axis_index("s")
        w = c * NS + s
        pltpu.sync_copy(row2_hbm.at[pl.ds(w * nch, nch)], ridx2)
        pltpu.sync_copy(col2_hbm.at[pl.ds(w * nch, nch)], cidx2)

        for q, u_hbm in enumerate((u0_hbm, u1_hbm)):
            pltpu.sync_copy(zrow_hbm, acc.at[pl.ds(s * rps, rps)])
            plsc.subcore_barrier()

            def start_g(g, b, u=u_hbm):
                pltpu.async_copy(u.at[ridx2.at[g]], bufs[b], gsems[b])

            def wait_g(g, b, u=u_hbm):
                pltpu.make_async_copy(u.at[ridx2.at[g]], bufs[b],
                                      gsems[b]).wait()

            def start_s(g, b):
                pltpu.async_copy(bufs[b], acc.at[cidx2.at[g]], ssems[b],
                                 add=True)

            def wait_s(g, b):
                pltpu.make_async_copy(bufs[b], acc.at[cidx2.at[g]],
                                      ssems[b]).wait()

            for b in range(3):
                start_g(b, b)
            for j in range(4):
                if j > 0:
                    wait_s(j - 1, (j + 3) % 4)
                start_g(j + 3, (j + 3) % 4)
                wait_g(j, j)
                start_s(j, j)

            def body(k, carry):
                for j in range(4):
                    g = 4 * k + j
                    bi = (j + 3) % 4
                    wait_s(g - 1, bi)
                    start_g(g + 3, bi)
                    wait_g(g, j)
                    start_s(g, j)
                return carry

            lax.fori_loop(1, nk - 1, body, 0)

            g0 = 4 * (nk - 1)
            for j in range(4):
                bi = (j + 3) % 4
                wait_s(g0 + j - 1, bi)
                if g0 + j + 3 <= nch - 1:
                    start_g(g0 + j + 3, bi)
                wait_g(g0 + j, j)
                start_s(g0 + j, j)

            wait_g(nch - 1, 0)
            start_s(nch - 1, 0)
            wait_s(nch - 2, 3)
            wait_s(nch - 1, 0)
            plsc.subcore_barrier()
            pltpu.sync_copy(
                acc.at[pl.ds(s * rps, rps)],
                out_hbm.at[pl.ds((c * NQ + q) * np_ + s * rps, rps)])
            plsc.subcore_barrier()

    return spmv


def _full_spec(shape):
    nd = len(shape)
    return pl.BlockSpec(shape, lambda i, _nd=nd: (0,) * _nd)


def _tc_call(body, out_widths, bn, np_, d, *args):
    grid = np_ // bn
    in_specs = []
    for a in args:
        if a.ndim == 4:
            in_specs.append(
                pl.BlockSpec((NC, NQ, bn, Q), lambda i: (0, 0, i, 0)))
        elif a.shape[0] == np_:
            in_specs.append(
                pl.BlockSpec((bn, a.shape[1]), lambda i: (i, 0)))
        else:
            in_specs.append(_full_spec(a.shape))
    out_shapes = tuple(
        jax.ShapeDtypeStruct((np_, w), jnp.float32) for w in out_widths)
    out_specs = tuple(
        pl.BlockSpec((bn, w), lambda i: (i, 0)) for w in out_widths)
    return pl.pallas_call(
        body,
        grid=(grid,),
        in_specs=in_specs,
        out_shape=out_shapes,
        out_specs=out_specs,
    )(*args)


def _sum_parts(sref):
    return jnp.concatenate(
        [sref[0, q] + sref[1, q] for q in range(NQ)], axis=1)


def _store_halves(u, urefs):
    for q, uref in enumerate(urefs):
        uref[...] = u[:, q * Q:(q + 1) * Q]


def _tc_dis(dparts, bn, np_):
    def body(dref, dis16_ref):
        deg = dref[0, :, 0:1] + dref[1, :, 0:1]
        dis = jnp.where(deg > 0.0, lax.rsqrt(jnp.maximum(deg, 1e-30)), 0.0)
        dis16_ref[...] = jnp.broadcast_to(dis, (bn, 16))

    return pl.pallas_call(
        body,
        grid=(np_ // bn,),
        in_specs=[pl.BlockSpec((NC, bn, 16), lambda i: (0, i, 0))],
        out_shape=jax.ShapeDtypeStruct((np_, 16), jnp.float32),
        out_specs=pl.BlockSpec((bn, 16), lambda i: (i, 0)),
    )(dparts)


def _tc_a(dis16, x, w0, bn, np_, d):
    def body(dref, xref, wref, uq0, uq1, acc_ref):
        dis = dref[:, 0:1]
        xb = xref[...]
        _store_halves(dis * xb, (uq0, uq1))
        acc_ref[...] = jnp.dot(xb, wref[...],
                               preferred_element_type=jnp.float32)
    return _tc_call(body, (Q, Q, d), bn, np_, d, dis16, x, w0)


def _tc_b(sparts, dis16, wk, acc, bn, np_, d):
    def body(sref, dref, wref, aref, uq0, uq1, acc_ref):
        dis = dref[:, 0:1]
        tx1 = -dis * _sum_parts(sref)
        _store_halves(dis * tx1, (uq0, uq1))
        acc_ref[...] = aref[...] + jnp.dot(
            tx1, wref[...], preferred_element_type=jnp.float32)
    return _tc_call(body, (Q, Q, d), bn, np_, d, sparts, dis16, wk, acc)


def _tc_e(sparts, dis16, tx0, acc, wk, b, rfl, bn, np_, d):
    def body(sref, dref, tref, aref, wref, bref, rref, out_ref):
        dis = dref[:, 0:1]
        tx2 = -2.0 * dis * _sum_parts(sref) - tref[...]
        o = aref[...] + jnp.dot(
            tx2, wref[...], preferred_element_type=jnp.float32) + bref[...]
        out_ref[...] = jnp.where(rref[...] > 0.0, jnp.maximum(o, 0.0), o)
    return _tc_call(body, (d,), bn, np_, d,
                    sparts, dis16, tx0, acc, wk, b, rfl)[0]


def kernel(x, edge_index, W1, b1, W2, b2):
    n, d = x.shape
    e = edge_index.shape[1]
    np_ = 10240
    bn = 1024
    rps = np_ // NS
    row2 = jnp.reshape(edge_index[0], (e // CH, CH))
    col2 = jnp.reshape(edge_index[1], (e // CH, CH))

    xp = jnp.pad(x, ((0, np_ - n), (0, 0)))
    zrow_q = jnp.zeros((rps, Q), jnp.float32)
    zrow_16 = jnp.zeros((rps, 16), jnp.float32)
    ones16 = jnp.ones((CH, 16), jnp.float32)
    ws = jnp.stack((W1, W2))
    bs = jnp.stack((jnp.reshape(b1, (1, d)), jnp.reshape(b2, (1, d))))
    rfls = jnp.stack((jnp.ones((1, d), jnp.float32),
                      jnp.zeros((1, d), jnp.float32)))

    deg_fn = _degree_kernel(np_, e)
    spmv_fn = _spmv_kernel(np_, e)

    def spmv(u0, u1):
        parts = spmv_fn(u0, u1, row2, col2, zrow_q)
        return jnp.reshape(parts, (NC, NQ, np_, Q))

    dparts = jnp.reshape(deg_fn(row2, ones16, zrow_16), (NC, np_, 16))
    dis16 = _tc_dis(dparts, bn, np_)

    def layer(x_in, wbr):
        wk, bk, rfl = wbr
        u0, u1, acc = _tc_a(dis16, x_in, wk[0], bn, np_, d)
        s1 = spmv(u0, u1)
        v0, v1, acc = _tc_b(s1, dis16, wk[1], acc, bn, np_, d)
        s2 = spmv(v0, v1)
        out = _tc_e(s2, dis16, x_in, acc, wk[2], bk, rfl, bn, np_, d)
        return out, None

    out, _ = lax.scan(layer, xp, (ws, bs, rfls))
    return out[:n]
